# baseline (speedup 1.0000x reference)
import functools

import jax
import jax.numpy as jnp
from jax import lax
from jax.experimental import pallas as pl
from jax.experimental.pallas import tpu as pltpu
from jax.experimental.pallas import tpu_sc as plsc

B = 1024
L = 512
D = 768
NC = 2
NS = 16
NW = NC * NS
N_TOK = B * L
LW = L // NW
CB = 2
TPC = CB * LW
N_CHUNK = B // CB


def _sc_body(x_hbm, seg_hbm, word_hbm, combo_hbm, out_hbm,
             xcol, scol, bases, rows0, rows1, idx0, idx1, sbuf,
             sg0, sg1, sw0, sw1):
    wid = lax.axis_index("s") * NC + lax.axis_index("c")
    l0 = wid * LW

    pltpu.sync_copy(x_hbm.at[wid], xcol)
    pltpu.sync_copy(seg_hbm.at[wid], scol)
    pltpu.sync_copy(combo_hbm.at[pl.ds(l0, LW)], bases.at[pl.ds(0, LW)])
    pltpu.sync_copy(combo_hbm.at[pl.ds(L + l0, LW)], bases.at[pl.ds(LW, LW)])

    bufs = ((rows0, idx0, sg0, sw0), (rows1, idx1, sg1, sw1))
    iota = lax.iota(jnp.int32, 16)

    def issue_gather(c, buf):
        rows, idxb, sg, _ = buf
        r, o = lax.div(c, 4), lax.rem(c, 4) * TPC
        for u in range(TPC // 16):
            idxb[pl.ds(u * 16, 16)] = xcol[r, pl.ds(o + u * 16, 16)]
        pltpu.async_copy(word_hbm.at[idxb], rows, sg)

    def wait_gather(buf):
        rows, idxb, sg, _ = buf
        pltpu.make_async_copy(word_hbm.at[idxb], rows, sg).wait()

    def issue_write(c, buf):
        rows, _, _, sw = buf
        b0 = c * CB
        for bi in range(CB):
            r0 = (b0 + bi) * L + l0
            pltpu.async_copy(rows.at[pl.ds(bi * 16, 16)],
                             out_hbm.at[pl.ds(r0, 16)], sw)

    def wait_write(buf):
        rows, _, _, sw = buf
        for bi in range(CB):
            pltpu.make_async_copy(rows.at[pl.ds(bi * 16, 16)],
                                  out_hbm.at[pl.ds(0, 16)], sw).wait()

    def accumulate(c, buf):
        rows, _, _, _ = buf
        r, o = lax.div(c, 4), lax.rem(c, 4) * TPC
        for u in range(TPC // 16):
            sbuf[pl.ds(u * 16, 16)] = scol[r, pl.ds(o + u * 16, 16)]

        def tbody(t, _):
            s = sbuf[pl.ds(t, 16)][0]
            li = lax.rem(t, LW)
            aoff = s * LW + li

            def dbody(j, _):
                plsc.addupdate(rows.at[t, pl.ds(j * 16, 16)],
                               bases[aoff, pl.ds(j * 16, 16)])
                return 0

            lax.fori_loop(0, D // 16, dbody, 0, unroll=8)
            return 0

        lax.fori_loop(0, TPC, tbody, 0)

    issue_gather(0, bufs[0])

    def pair(i, carry):
        c0 = 2 * i
        for p in range(2):
            c = c0 + p
            cur, nxt = bufs[p], bufs[1 - p]

            @pl.when(c >= 1)
            def _():
                wait_write(nxt)

            @pl.when(c + 1 < N_CHUNK)
            def _():
                issue_gather(c + 1, nxt)

            wait_gather(cur)
            accumulate(c, cur)
            issue_write(c, cur)
        return carry

    lax.fori_loop(0, N_CHUNK // 2, pair, 0)
    wait_write(bufs[1])


@functools.partial(jax.jit, static_argnames=())
def _sc_embed(x, seg, word_table, combo):
    mesh = plsc.VectorSubcoreMesh(core_axis_name="c", subcore_axis_name="s",
                                  num_cores=NC, num_subcores=NS)
    f = pl.kernel(
        _sc_body,
        out_type=jax.ShapeDtypeStruct((N_TOK, D), jnp.float32),
        mesh=mesh,
        scratch_types=[
            pltpu.VMEM((128, 128), jnp.int32),
            pltpu.VMEM((128, 128), jnp.int32),
            pltpu.VMEM((2 * LW, D), jnp.float32),
            pltpu.VMEM((TPC, D), jnp.float32),
            pltpu.VMEM((TPC, D), jnp.float32),
            pltpu.VMEM((TPC,), jnp.int32),
            pltpu.VMEM((TPC,), jnp.int32),
            pltpu.VMEM((TPC + 16,), jnp.int32),
            pltpu.SemaphoreType.DMA,
            pltpu.SemaphoreType.DMA,
            pltpu.SemaphoreType.DMA,
            pltpu.SemaphoreType.DMA,
        ],
    )
    return f(x, seg, word_table, combo)


def _mask_body(x_ref, o_ref):
    o_ref[...] = x_ref[...] > 0


def _mask(x):
    return pl.pallas_call(
        _mask_body,
        out_shape=jax.ShapeDtypeStruct((B, L), jnp.bool_),
        grid=(8,),
        in_specs=[pl.BlockSpec((B // 8, L), lambda i: (i, 0))],
        out_specs=pl.BlockSpec((B // 8, L), lambda i: (i, 0)),
    )(x)


def kernel(x, seg, word_table, pos_table, seg_table):
    combo = (seg_table[:, None, :] + pos_table[None, :, :]).reshape(2 * L, D)
    xw = x.reshape(B, NW, LW).transpose(1, 0, 2).reshape(NW, 128, 128)
    sw = seg.reshape(B, NW, LW).transpose(1, 0, 2).reshape(NW, 128, 128)
    out_flat = _sc_embed(xw, sw, word_table, combo)
    return out_flat.reshape(B, L, D), _mask(x)


# speedup vs baseline: 1.4867x; 1.4867x over previous
import functools

import jax
import jax.numpy as jnp
from jax import lax
from jax.experimental import pallas as pl
from jax.experimental.pallas import tpu as pltpu
from jax.experimental.pallas import tpu_sc as plsc

B = 1024
L = 512
D = 768
NC = 2
NS = 16
NW = NC * NS
N_TOK = B * L
TOK_PER_W = N_TOK // NW
C = 32
N_CHUNK = TOK_PER_W // C
DSL = D // 16


def _sc_body(x_hbm, seg_hbm, word_hbm, combo_hbm, out_hbm,
             rows0, rows1, add0, add1, idx0, idx1, cid0, cid1, sv0, sv1,
             gw0, gw1, gc0, gc1, wr0, wr1):
    wid = lax.axis_index("s") * NC + lax.axis_index("c")
    base = wid * TOK_PER_W
    iota = lax.iota(jnp.int32, 16)

    bufs = ((rows0, add0, idx0, cid0, sv0, gw0, gc0, wr0),
            (rows1, add1, idx1, cid1, sv1, gw1, gc1, wr1))

    def issue_in(c, buf):
        rows, addv, idxb, cidx, segv, gw, gc, _ = buf
        t0 = base + c * C
        pltpu.sync_copy(x_hbm.at[pl.ds(t0, C)], idxb)
        pltpu.sync_copy(seg_hbm.at[pl.ds(t0, C)], segv)
        pltpu.async_copy(word_hbm.at[idxb], rows, gw)
        p0 = lax.rem(c * C, L)
        for u in range(C // 16):
            s16 = segv[pl.ds(u * 16, 16)]
            cidx[pl.ds(u * 16, 16)] = s16 * L + (iota + (p0 + u * 16))
        pltpu.async_copy(combo_hbm.at[cidx], addv, gc)

    def wait_in(buf):
        rows, addv, idxb, cidx, _, gw, gc, _ = buf
        pltpu.make_async_copy(word_hbm.at[idxb], rows, gw).wait()
        pltpu.make_async_copy(combo_hbm.at[cidx], addv, gc).wait()

    def accumulate(buf):
        rows, addv = buf[0], buf[1]

        def per_row(ci, _):
            for j in range(DSL):
                plsc.addupdate(rows.at[ci, pl.ds(j * 16, 16)],
                               addv[ci, pl.ds(j * 16, 16)])
            return 0

        lax.fori_loop(0, C, per_row, 0)

    def issue_out(c, buf):
        rows, wr = buf[0], buf[7]
        pltpu.async_copy(rows, out_hbm.at[pl.ds(base + c * C, C)], wr)

    def wait_out(buf):
        rows, wr = buf[0], buf[7]
        pltpu.make_async_copy(rows, out_hbm.at[pl.ds(base, C)], wr).wait()

    issue_in(0, bufs[0])

    def pair(i, carry):
        c0 = 2 * i
        for p in range(2):
            c = c0 + p
            cur, nxt = bufs[p], bufs[1 - p]

            @pl.when(c >= 1)
            def _():
                wait_out(nxt)

            @pl.when(c + 1 < N_CHUNK)
            def _():
                issue_in(c + 1, nxt)

            wait_in(cur)
            accumulate(cur)
            issue_out(c, cur)
        return carry

    lax.fori_loop(0, N_CHUNK // 2, pair, 0)
    wait_out(bufs[1])


@functools.partial(jax.jit, static_argnames=())
def _sc_embed(x_flat, seg_flat, word_table, combo):
    mesh = plsc.VectorSubcoreMesh(core_axis_name="c", subcore_axis_name="s",
                                  num_cores=NC, num_subcores=NS)
    f = pl.kernel(
        _sc_body,
        out_type=jax.ShapeDtypeStruct((N_TOK, D), jnp.float32),
        mesh=mesh,
        scratch_types=[
            pltpu.VMEM((C, D), jnp.float32),
            pltpu.VMEM((C, D), jnp.float32),
            pltpu.VMEM((C, D), jnp.float32),
            pltpu.VMEM((C, D), jnp.float32),
            pltpu.VMEM((C,), jnp.int32),
            pltpu.VMEM((C,), jnp.int32),
            pltpu.VMEM((C,), jnp.int32),
            pltpu.VMEM((C,), jnp.int32),
            pltpu.VMEM((C,), jnp.int32),
            pltpu.VMEM((C,), jnp.int32),
            pltpu.SemaphoreType.DMA,
            pltpu.SemaphoreType.DMA,
            pltpu.SemaphoreType.DMA,
            pltpu.SemaphoreType.DMA,
            pltpu.SemaphoreType.DMA,
            pltpu.SemaphoreType.DMA,
        ],
    )
    return f(x_flat, seg_flat, word_table, combo)


def _mask_body(x_ref, o_ref):
    o_ref[...] = x_ref[...] > 0


def _mask(x):
    return pl.pallas_call(
        _mask_body,
        out_shape=jax.ShapeDtypeStruct((B, L), jnp.bool_),
        grid=(8,),
        in_specs=[pl.BlockSpec((B // 8, L), lambda i: (i, 0))],
        out_specs=pl.BlockSpec((B // 8, L), lambda i: (i, 0)),
    )(x)


def kernel(x, seg, word_table, pos_table, seg_table):
    combo = (seg_table[:, None, :] + pos_table[None, :, :]).reshape(2 * L, D)
    out_flat = _sc_embed(x.reshape(N_TOK), seg.reshape(N_TOK), word_table, combo)
    return out_flat.reshape(B, L, D), _mask(x)
